# Initial kernel scaffold; baseline (speedup 1.0000x reference)
import jax
import jax.numpy as jnp
from jax.experimental import pallas as pl


def kernel(h, e, edge_index, A_w, A_b, B_w, B_b, C_w, C_b, node_w, node_b, bn_h_g, bn_h_b, bn_e_g, bn_e_b):
    raise NotImplementedError("write your pallas kernel here")


# speedup vs baseline: 1.2921x; 1.2921x over previous
import functools

import jax
import jax.numpy as jnp
from jax import lax
from jax.experimental import pallas as pl
from jax.experimental.pallas import tpu as pltpu
from jax.experimental.pallas import tpu_sc as plsc

NC = 2
NS = 16
NW = NC * NS

N = 10000
E = 320000
D = 128

R_BLK = 1280
E_GRID = E // R_BLK

IDX_CHUNK = 128
N_IDX_ROWS = E // IDX_CHUNK

ROWS_PER_W = 320
NPAD = NW * ROWS_PER_W
S_STRIP = 6400
N_STRIP = E // S_STRIP
GB = 128


def _sc_mesh():
    return plsc.VectorSubcoreMesh(
        core_axis_name="c", subcore_axis_name="s", num_cores=NC, num_subcores=NS
    )


def _wid():
    return lax.axis_index("s") * NC + lax.axis_index("c")


def _node_proj_kernel(h_ref, aw_ref, ab_ref, bw_ref, bb_ref, tb_ref, bh_ref):
    h = h_ref[...]
    dn = (((1,), (1,)), ((), ()))
    ah = lax.dot_general(h, aw_ref[...], dn, preferred_element_type=jnp.float32)
    bh = lax.dot_general(h, bw_ref[...], dn, preferred_element_type=jnp.float32)
    ah = ah + ab_ref[...]
    bh = bh + bb_ref[...]
    tb_ref[:, 0:D] = bh
    tb_ref[:, D : 2 * D] = ah
    bh_ref[...] = bh


def _ce_kernel(e_ref, cw_ref, cb_ref, ce_ref):
    dn = (((1,), (1,)), ((), ()))
    ce_ref[...] = (
        lax.dot_general(e_ref[...], cw_ref[...], dn, preferred_element_type=jnp.float32)
        + cb_ref[...]
    )


def _edge_math_kernel(ce_ref, g1_ref, g2_ref, r_ref, msg_ref, stats_ref, acc_ref):
    i = pl.program_id(0)
    e_ij = ce_ref[...] + g1_ref[:, 0:D] + g2_ref[...]
    r = jnp.maximum(e_ij, 0.0)
    msg = jnp.maximum(jax.nn.sigmoid(e_ij) * g1_ref[:, D : 2 * D], 0.0)
    r_ref[...] = r
    msg_ref[...] = msg
    blk = jnp.concatenate(
        [jnp.sum(r, axis=0, keepdims=True), jnp.sum(r * r, axis=0, keepdims=True)],
        axis=0,
    )

    @pl.when(i == 0)
    def _():
        acc_ref[...] = blk

    @pl.when(i > 0)
    def _():
        acc_ref[...] = acc_ref[...] + blk

    @pl.when(i == pl.num_programs(0) - 1)
    def _():
        stats_ref[...] = acc_ref[...]


def _e_out_kernel(e_ref, r_ref, stats_ref, g_ref, b_ref, out_ref):
    mu = stats_ref[0:1, :] * (1.0 / E)
    var = stats_ref[1:2, :] * (1.0 / E) - mu * mu
    scale = lax.rsqrt(var + 1e-5) * g_ref[...]
    out_ref[...] = e_ref[...] + (r_ref[...] - mu) * scale + b_ref[...]


def _node_out_kernel(h_ref, c_ref, w1_ref, w2_ref, nb_ref, g_ref, b_ref, out_ref):
    h = h_ref[...]
    c = c_ref[0:N, :]
    dn = (((1,), (1,)), ((), ()))
    bundle = (
        lax.dot_general(h, w1_ref[...], dn, preferred_element_type=jnp.float32)
        + lax.dot_general(c, w2_ref[...], dn, preferred_element_type=jnp.float32)
        + nb_ref[...]
    )
    nrm = jnp.sqrt(jnp.sum(bundle * bundle, axis=1, keepdims=True))
    bundle = bundle / jnp.maximum(nrm, 1e-12)
    x = jnp.maximum(bundle, 0.0)
    mu = jnp.mean(x, axis=0, keepdims=True)
    var = jnp.mean((x - mu) ** 2, axis=0, keepdims=True)
    out_ref[...] = h + (x - mu) * lax.rsqrt(var + 1e-5) * g_ref[...] + b_ref[...]


def _sc_gather_body(
    src_hbm, dst_hbm, tb_hbm, bh_hbm, g1_hbm, g2_hbm, sidx, didx, g1b, g2b, sem1, sem2
):
    w = _wid()
    ntrips = (N_IDX_ROWS - w + NW - 1) // NW

    def body(t, carry):
        base = (w + t * NW) * IDX_CHUNK
        pltpu.sync_copy(src_hbm.at[pl.ds(base, IDX_CHUNK)], sidx)
        pltpu.sync_copy(dst_hbm.at[pl.ds(base, IDX_CHUNK)], didx)
        c1 = pltpu.async_copy(tb_hbm.at[sidx], g1b, sem1)
        c2 = pltpu.async_copy(bh_hbm.at[didx], g2b, sem2)
        c1.wait()
        c2.wait()
        pltpu.sync_copy(g1b, g1_hbm.at[pl.ds(base, IDX_CHUNK)])
        pltpu.sync_copy(g2b, g2_hbm.at[pl.ds(base, IDX_CHUNK)])
        return carry

    lax.fori_loop(0, ntrips, body, 0)


def _sc_segmax_body(dst_hbm, msg_hbm, c_hbm, ltab, dstbuf, ceid, cdst, rowsb, sem):
    w = _wid()
    lo = w * ROWS_PER_W
    hi = lo + ROWS_PER_W

    zf = jnp.zeros((16,), jnp.float32)
    zi = jnp.zeros((16,), jnp.int32)

    def zrow(i, carry):
        for k in range(D // 16):
            ltab[i, pl.ds(k * 16, 16)] = zf
        return carry

    lax.fori_loop(0, ROWS_PER_W, zrow, 0)

    def zidx(i, carry):
        ceid[pl.ds(i * 16, 16)] = zi
        return carry

    lax.fori_loop(0, (S_STRIP + 16) // 16, zidx, 0)

    lane = lax.iota(jnp.int32, 16)

    def strip(s, carry):
        sbase = s * S_STRIP
        pltpu.sync_copy(dst_hbm.at[pl.ds(sbase, S_STRIP)], dstbuf)

        def scan(j, cur):
            d = dstbuf[pl.ds(j * 16, 16)]
            m = (d >= lo) & (d < hi)
            eid = sbase + j * 16 + lane
            mi = m.astype(jnp.int32)
            pref = plsc.cumsum(mi)
            pos = cur + pref - 1
            plsc.store_scatter(ceid, [pos], eid, mask=m)
            plsc.store_scatter(cdst, [pos], d, mask=m)
            return cur + jnp.sum(mi)

        cur = lax.fori_loop(0, S_STRIP // 16, scan, 0)
        nb = (cur + GB - 1) // GB

        def batch(b, carry):
            pltpu.async_copy(msg_hbm.at[ceid.at[pl.ds(b * GB, GB)]], rowsb, sem).wait()
            vb = jnp.minimum(GB, cur - b * GB)

            def upd(t, c2):
                dsp = plsc.load_gather(cdst, [jnp.full((16,), b * GB + t, jnp.int32)])
                row = dsp - lo
                for k in range(D // 16):
                    col = k * 16 + lane
                    old = plsc.load_gather(ltab, [row, col])
                    new = jnp.maximum(old, rowsb[t, pl.ds(k * 16, 16)])
                    plsc.store_scatter(ltab, [row, col], new)
                return c2

            lax.fori_loop(0, vb, upd, 0)
            return carry

        lax.fori_loop(0, nb, batch, 0)
        return carry

    lax.fori_loop(0, N_STRIP, strip, 0)
    pltpu.sync_copy(ltab, c_hbm.at[pl.ds(lo, ROWS_PER_W)])


def kernel(h, e, edge_index, A_w, A_b, B_w, B_b, C_w, C_b, node_w, node_b,
           bn_h_g, bn_h_b, bn_e_g, bn_e_b):
    f32 = jnp.float32
    src = edge_index[0].astype(jnp.int32)
    dst = edge_index[1].astype(jnp.int32)

    tb, bh = pl.pallas_call(
        _node_proj_kernel,
        out_shape=[
            jax.ShapeDtypeStruct((N, 2 * D), f32),
            jax.ShapeDtypeStruct((N, D), f32),
        ],
    )(h, A_w, A_b.reshape(1, D), B_w, B_b.reshape(1, D))

    ce = pl.pallas_call(
        _ce_kernel,
        grid=(E_GRID,),
        in_specs=[
            pl.BlockSpec((R_BLK, D), lambda i: (i, 0)),
            pl.BlockSpec((D, D), lambda i: (0, 0)),
            pl.BlockSpec((1, D), lambda i: (0, 0)),
        ],
        out_specs=pl.BlockSpec((R_BLK, D), lambda i: (i, 0)),
        out_shape=jax.ShapeDtypeStruct((E, D), f32),
    )(e, C_w, C_b.reshape(1, D))

    gather_fn = pl.kernel(
        _sc_gather_body,
        out_type=[
            jax.ShapeDtypeStruct((E, 2 * D), f32),
            jax.ShapeDtypeStruct((E, D), f32),
        ],
        mesh=_sc_mesh(),
        scratch_types=[
            pltpu.VMEM((IDX_CHUNK,), jnp.int32),
            pltpu.VMEM((IDX_CHUNK,), jnp.int32),
            pltpu.VMEM((IDX_CHUNK, 2 * D), f32),
            pltpu.VMEM((IDX_CHUNK, D), f32),
            pltpu.SemaphoreType.DMA,
            pltpu.SemaphoreType.DMA,
        ],
    )
    g1, g2 = gather_fn(src, dst, tb, bh)

    r, msg, e_stats = pl.pallas_call(
        _edge_math_kernel,
        grid=(E_GRID,),
        in_specs=[
            pl.BlockSpec((R_BLK, D), lambda i: (i, 0)),
            pl.BlockSpec((R_BLK, 2 * D), lambda i: (i, 0)),
            pl.BlockSpec((R_BLK, D), lambda i: (i, 0)),
        ],
        out_specs=[
            pl.BlockSpec((R_BLK, D), lambda i: (i, 0)),
            pl.BlockSpec((R_BLK, D), lambda i: (i, 0)),
            pl.BlockSpec((2, D), lambda i: (0, 0)),
        ],
        out_shape=[
            jax.ShapeDtypeStruct((E, D), f32),
            jax.ShapeDtypeStruct((E, D), f32),
            jax.ShapeDtypeStruct((2, D), f32),
        ],
        scratch_shapes=[pltpu.VMEM((2, D), f32)],
    )(ce, g1, g2)

    segmax_fn = pl.kernel(
        _sc_segmax_body,
        out_type=jax.ShapeDtypeStruct((NPAD, D), f32),
        mesh=_sc_mesh(),
        compiler_params=pltpu.CompilerParams(needs_layout_passes=False),
        scratch_types=[
            pltpu.VMEM((ROWS_PER_W, D), f32),
            pltpu.VMEM((S_STRIP,), jnp.int32),
            pltpu.VMEM((S_STRIP + 16, ), jnp.int32),
            pltpu.VMEM((S_STRIP + 16, ), jnp.int32),
            pltpu.VMEM((GB, D), f32),
            pltpu.SemaphoreType.DMA,
        ],
    )
    c = segmax_fn(dst, msg)

    e_out = pl.pallas_call(
        _e_out_kernel,
        grid=(E_GRID,),
        in_specs=[
            pl.BlockSpec((R_BLK, D), lambda i: (i, 0)),
            pl.BlockSpec((R_BLK, D), lambda i: (i, 0)),
            pl.BlockSpec((2, D), lambda i: (0, 0)),
            pl.BlockSpec((1, D), lambda i: (0, 0)),
            pl.BlockSpec((1, D), lambda i: (0, 0)),
        ],
        out_specs=pl.BlockSpec((R_BLK, D), lambda i: (i, 0)),
        out_shape=jax.ShapeDtypeStruct((E, D), f32),
    )(e, r, e_stats, bn_e_g.reshape(1, D), bn_e_b.reshape(1, D))

    h_out = pl.pallas_call(
        _node_out_kernel,
        out_shape=jax.ShapeDtypeStruct((N, D), f32),
    )(
        h,
        c,
        node_w[:, 0:D],
        node_w[:, D : 2 * D],
        node_b.reshape(1, D),
        bn_h_g.reshape(1, D),
        bn_h_b.reshape(1, D),
    )

    return (h_out, e_out)


# speedup vs baseline: 1.3123x; 1.0157x over previous
import functools

import jax
import jax.numpy as jnp
from jax import lax
from jax.experimental import pallas as pl
from jax.experimental.pallas import tpu as pltpu
from jax.experimental.pallas import tpu_sc as plsc

NC = 2
NS = 16
NW = NC * NS

N = 10000
E = 320000
D = 128

R_BLK = 1280
E_GRID = E // R_BLK

IDX_CHUNK = 128
N_IDX_ROWS = E // IDX_CHUNK

ROWS_PER_W = 320
NPAD = NW * ROWS_PER_W
S_STRIP = 6400
N_STRIP = E // S_STRIP
GB = 128


def _sc_mesh():
    return plsc.VectorSubcoreMesh(
        core_axis_name="c", subcore_axis_name="s", num_cores=NC, num_subcores=NS
    )


def _wid():
    return lax.axis_index("s") * NC + lax.axis_index("c")


def _node_proj_kernel(h_ref, aw_ref, ab_ref, bw_ref, bb_ref, tb_ref, bh_ref):
    h = h_ref[...]
    dn = (((1,), (1,)), ((), ()))
    ah = lax.dot_general(h, aw_ref[...], dn, preferred_element_type=jnp.float32)
    bh = lax.dot_general(h, bw_ref[...], dn, preferred_element_type=jnp.float32)
    ah = ah + ab_ref[...]
    bh = bh + bb_ref[...]
    tb_ref[:, 0:D] = bh
    tb_ref[:, D : 2 * D] = ah
    bh_ref[...] = bh


def _ce_kernel(e_ref, cw_ref, cb_ref, ce_ref):
    dn = (((1,), (1,)), ((), ()))
    ce_ref[...] = (
        lax.dot_general(e_ref[...], cw_ref[...], dn, preferred_element_type=jnp.float32)
        + cb_ref[...]
    )


def _edge_math_kernel(ce_ref, g1_ref, g2_ref, r_ref, msg_ref, stats_ref, acc_ref):
    i = pl.program_id(0)
    e_ij = ce_ref[...] + g1_ref[:, 0:D] + g2_ref[...]
    r = jnp.maximum(e_ij, 0.0)
    msg = jnp.maximum(jax.nn.sigmoid(e_ij) * g1_ref[:, D : 2 * D], 0.0)
    r_ref[...] = r
    msg_ref[...] = msg
    blk = jnp.concatenate(
        [jnp.sum(r, axis=0, keepdims=True), jnp.sum(r * r, axis=0, keepdims=True)],
        axis=0,
    )

    @pl.when(i == 0)
    def _():
        acc_ref[...] = blk

    @pl.when(i > 0)
    def _():
        acc_ref[...] = acc_ref[...] + blk

    @pl.when(i == pl.num_programs(0) - 1)
    def _():
        stats_ref[...] = acc_ref[...]


def _e_out_kernel(e_ref, r_ref, stats_ref, g_ref, b_ref, out_ref):
    mu = stats_ref[0:1, :] * (1.0 / E)
    var = stats_ref[1:2, :] * (1.0 / E) - mu * mu
    scale = lax.rsqrt(var + 1e-5) * g_ref[...]
    out_ref[...] = e_ref[...] + (r_ref[...] - mu) * scale + b_ref[...]


def _node_out_kernel(h_ref, c_ref, w1_ref, w2_ref, nb_ref, g_ref, b_ref, out_ref):
    h = h_ref[...]
    c = c_ref[0:N, :]
    dn = (((1,), (1,)), ((), ()))
    bundle = (
        lax.dot_general(h, w1_ref[...], dn, preferred_element_type=jnp.float32)
        + lax.dot_general(c, w2_ref[...], dn, preferred_element_type=jnp.float32)
        + nb_ref[...]
    )
    nrm = jnp.sqrt(jnp.sum(bundle * bundle, axis=1, keepdims=True))
    bundle = bundle / jnp.maximum(nrm, 1e-12)
    x = jnp.maximum(bundle, 0.0)
    mu = jnp.mean(x, axis=0, keepdims=True)
    var = jnp.mean((x - mu) ** 2, axis=0, keepdims=True)
    out_ref[...] = h + (x - mu) * lax.rsqrt(var + 1e-5) * g_ref[...] + b_ref[...]


def _sc_gather_body(
    src_hbm, dst_hbm, tb_hbm, bh_hbm, g1_hbm, g2_hbm, sidx, didx, g1b, g2b, sem1, sem2
):
    w = _wid()
    ntrips = (N_IDX_ROWS - w + NW - 1) // NW

    def body(t, carry):
        base = (w + t * NW) * IDX_CHUNK
        pltpu.sync_copy(src_hbm.at[pl.ds(base, IDX_CHUNK)], sidx)
        pltpu.sync_copy(dst_hbm.at[pl.ds(base, IDX_CHUNK)], didx)
        c1 = pltpu.async_copy(tb_hbm.at[sidx], g1b, sem1)
        c2 = pltpu.async_copy(bh_hbm.at[didx], g2b, sem2)
        c1.wait()
        c2.wait()
        pltpu.sync_copy(g1b, g1_hbm.at[pl.ds(base, IDX_CHUNK)])
        pltpu.sync_copy(g2b, g2_hbm.at[pl.ds(base, IDX_CHUNK)])
        return carry

    lax.fori_loop(0, ntrips, body, 0)


def _sc_segmax_body(dst_hbm, msg_hbm, c_hbm, ltab_a, ltab_b, dstbuf, ceid, cdst, rowsb, sem):
    w = _wid()
    lo = w * ROWS_PER_W
    hi = lo + ROWS_PER_W

    zf = jnp.zeros((16,), jnp.float32)
    zi = jnp.zeros((16,), jnp.int32)
    neg1 = jnp.full((16,), -1, jnp.int32)

    def zrow(i, carry):
        for k in range(D // 16):
            ltab_a[i, pl.ds(k * 16, 16)] = zf
            ltab_b[i, pl.ds(k * 16, 16)] = zf
        return carry

    lax.fori_loop(0, ROWS_PER_W, zrow, 0)

    def zidx(i, carry):
        ceid[pl.ds(i * 16, 16)] = zi
        return carry

    lax.fori_loop(0, (S_STRIP + 16) // 16, zidx, 0)

    lane = lax.iota(jnp.int32, 16)
    last = jnp.full((16,), 15, jnp.int32)

    def strip(s, carry):
        sbase = s * S_STRIP
        pltpu.sync_copy(dst_hbm.at[pl.ds(sbase, S_STRIP)], dstbuf)

        def scan(j, curv):
            d0 = dstbuf[pl.ds(j * 32, 16)]
            d1 = dstbuf[pl.ds(j * 32 + 16, 16)]
            m0 = (d0 >= lo) & (d0 < hi)
            m1 = (d1 >= lo) & (d1 < hi)
            p0 = plsc.cumsum(m0.astype(jnp.int32))
            p1 = plsc.cumsum(m1.astype(jnp.int32))
            c0 = p0.at[last].get(mode="promise_in_bounds")
            pos0 = curv + p0 - 1
            pos1 = curv + c0 + p1 - 1
            eid0 = sbase + j * 32 + lane
            plsc.store_scatter(ceid, [pos0], eid0, mask=m0)
            plsc.store_scatter(cdst, [pos0], d0, mask=m0)
            plsc.store_scatter(ceid, [pos1], eid0 + 16, mask=m1)
            plsc.store_scatter(cdst, [pos1], d1, mask=m1)
            c1 = p1.at[last].get(mode="promise_in_bounds")
            return curv + c0 + c1

        curv = lax.fori_loop(0, S_STRIP // 32, scan, jnp.zeros((16,), jnp.int32))
        cur = jnp.max(curv)
        curvs = jnp.zeros((16,), jnp.int32) + cur
        for t in range(GB // 16):
            plsc.store_scatter(cdst, [curvs + t * 16 + lane], neg1)
        nb = (cur + GB - 1) // GB

        def batch(b, carry):
            pltpu.async_copy(msg_hbm.at[ceid.at[pl.ds(b * GB, GB)]], rowsb, sem).wait()

            def pair(g, c3):
                for bank, ge in ((ltab_a, 2 * g), (ltab_b, 2 * g + 1)):
                    dvec = cdst[pl.ds(b * GB + ge * 16, 16)]
                    rows16 = dvec - lo
                    for i in range(16):
                        row = rows16.at[jnp.full((16,), i, jnp.int32)].get(mode="promise_in_bounds")
                        m = (row >= 0) & (row < ROWS_PER_W)
                        for k in range(D // 16):
                            col = k * 16 + lane
                            old = plsc.load_gather(bank, [row, col], mask=m)
                            new = jnp.maximum(old, rowsb[ge * 16 + i, pl.ds(k * 16, 16)])
                            plsc.store_scatter(bank, [row, col], new, mask=m)
                return c3

            lax.fori_loop(0, GB // 32, pair, 0)
            return carry

        lax.fori_loop(0, nb, batch, 0)
        return carry

    lax.fori_loop(0, N_STRIP, strip, 0)

    def merge(i, carry):
        for k in range(D // 16):
            sl = pl.ds(k * 16, 16)
            ltab_a[i, sl] = jnp.maximum(ltab_a[i, sl], ltab_b[i, sl])
        return carry

    lax.fori_loop(0, ROWS_PER_W, merge, 0)
    pltpu.sync_copy(ltab_a, c_hbm.at[pl.ds(lo, ROWS_PER_W)])


def kernel(h, e, edge_index, A_w, A_b, B_w, B_b, C_w, C_b, node_w, node_b,
           bn_h_g, bn_h_b, bn_e_g, bn_e_b):
    f32 = jnp.float32
    src = edge_index[0].astype(jnp.int32)
    dst = edge_index[1].astype(jnp.int32)

    tb, bh = pl.pallas_call(
        _node_proj_kernel,
        out_shape=[
            jax.ShapeDtypeStruct((N, 2 * D), f32),
            jax.ShapeDtypeStruct((N, D), f32),
        ],
    )(h, A_w, A_b.reshape(1, D), B_w, B_b.reshape(1, D))

    ce = pl.pallas_call(
        _ce_kernel,
        grid=(E_GRID,),
        in_specs=[
            pl.BlockSpec((R_BLK, D), lambda i: (i, 0)),
            pl.BlockSpec((D, D), lambda i: (0, 0)),
            pl.BlockSpec((1, D), lambda i: (0, 0)),
        ],
        out_specs=pl.BlockSpec((R_BLK, D), lambda i: (i, 0)),
        out_shape=jax.ShapeDtypeStruct((E, D), f32),
    )(e, C_w, C_b.reshape(1, D))

    gather_fn = pl.kernel(
        _sc_gather_body,
        out_type=[
            jax.ShapeDtypeStruct((E, 2 * D), f32),
            jax.ShapeDtypeStruct((E, D), f32),
        ],
        mesh=_sc_mesh(),
        scratch_types=[
            pltpu.VMEM((IDX_CHUNK,), jnp.int32),
            pltpu.VMEM((IDX_CHUNK,), jnp.int32),
            pltpu.VMEM((IDX_CHUNK, 2 * D), f32),
            pltpu.VMEM((IDX_CHUNK, D), f32),
            pltpu.SemaphoreType.DMA,
            pltpu.SemaphoreType.DMA,
        ],
    )
    g1, g2 = gather_fn(src, dst, tb, bh)

    r, msg, e_stats = pl.pallas_call(
        _edge_math_kernel,
        grid=(E_GRID,),
        in_specs=[
            pl.BlockSpec((R_BLK, D), lambda i: (i, 0)),
            pl.BlockSpec((R_BLK, 2 * D), lambda i: (i, 0)),
            pl.BlockSpec((R_BLK, D), lambda i: (i, 0)),
        ],
        out_specs=[
            pl.BlockSpec((R_BLK, D), lambda i: (i, 0)),
            pl.BlockSpec((R_BLK, D), lambda i: (i, 0)),
            pl.BlockSpec((2, D), lambda i: (0, 0)),
        ],
        out_shape=[
            jax.ShapeDtypeStruct((E, D), f32),
            jax.ShapeDtypeStruct((E, D), f32),
            jax.ShapeDtypeStruct((2, D), f32),
        ],
        scratch_shapes=[pltpu.VMEM((2, D), f32)],
    )(ce, g1, g2)

    segmax_fn = pl.kernel(
        _sc_segmax_body,
        out_type=jax.ShapeDtypeStruct((NPAD, D), f32),
        mesh=_sc_mesh(),
        compiler_params=pltpu.CompilerParams(needs_layout_passes=False),
        scratch_types=[
            pltpu.VMEM((ROWS_PER_W, D), f32),
            pltpu.VMEM((ROWS_PER_W, D), f32),
            pltpu.VMEM((S_STRIP,), jnp.int32),
            pltpu.VMEM((S_STRIP + 16, ), jnp.int32),
            pltpu.VMEM((S_STRIP + GB + 16, ), jnp.int32),
            pltpu.VMEM((GB, D), f32),
            pltpu.SemaphoreType.DMA,
        ],
    )
    c = segmax_fn(dst, msg)

    e_out = pl.pallas_call(
        _e_out_kernel,
        grid=(E_GRID,),
        in_specs=[
            pl.BlockSpec((R_BLK, D), lambda i: (i, 0)),
            pl.BlockSpec((R_BLK, D), lambda i: (i, 0)),
            pl.BlockSpec((2, D), lambda i: (0, 0)),
            pl.BlockSpec((1, D), lambda i: (0, 0)),
            pl.BlockSpec((1, D), lambda i: (0, 0)),
        ],
        out_specs=pl.BlockSpec((R_BLK, D), lambda i: (i, 0)),
        out_shape=jax.ShapeDtypeStruct((E, D), f32),
    )(e, r, e_stats, bn_e_g.reshape(1, D), bn_e_b.reshape(1, D))

    h_out = pl.pallas_call(
        _node_out_kernel,
        out_shape=jax.ShapeDtypeStruct((N, D), f32),
    )(
        h,
        c,
        node_w[:, 0:D],
        node_w[:, D : 2 * D],
        node_b.reshape(1, D),
        bn_h_g.reshape(1, D),
        bn_h_b.reshape(1, D),
    )

    return (h_out, e_out)
